# baseline (speedup 1.0000x reference)
import functools

import jax
import jax.numpy as jnp
from jax import lax
from jax.experimental import pallas as pl
from jax.experimental.pallas import tpu as pltpu
from jax.experimental.pallas import tpu_sc as plsc

_SCALE = 11.313708498984761
_D = 128
_C = 128


@functools.partial(jax.jit, static_argnums=(0,))
def _embed(n_rows, idx, table):
    info = plsc.get_sparse_core_info()
    num_cores, num_subcores = info.num_cores, info.num_subcores
    nw = num_cores * num_subcores
    b_per_w = n_rows // nw
    g_chunks = b_per_w // _C

    mesh = plsc.VectorSubcoreMesh(core_axis_name="c", subcore_axis_name="s")

    @functools.partial(
        pl.kernel,
        mesh=mesh,
        out_type=jax.ShapeDtypeStruct((n_rows, _D), jnp.float32),
        scratch_types=[
            pltpu.VMEM((g_chunks, _C), jnp.int32),
            pltpu.VMEM((_C, _D), jnp.float32),
            pltpu.VMEM((_C, _D), jnp.float32),
            pltpu.SemaphoreType.DMA,
            pltpu.SemaphoreType.DMA,
            pltpu.SemaphoreType.DMA,
            pltpu.SemaphoreType.DMA,
        ],
    )
    def k(idx_hbm, table_hbm, out_hbm, idx_v, rows0, rows1, g0, g1, o0, o1):
        wid = lax.axis_index("s") * num_cores + lax.axis_index("c")
        base = wid * b_per_w
        pltpu.sync_copy(idx_hbm.at[wid], idx_v)

        bufs = (rows0, rows1)
        gsems = (g0, g1)
        osems = (o0, o1)

        def start_gather(g, b):
            pltpu.async_copy(table_hbm.at[idx_v.at[g]], bufs[b], gsems[b])

        def wait_gather(g, b):
            pltpu.make_async_copy(
                table_hbm.at[idx_v.at[g]], bufs[b], gsems[b]
            ).wait()

        def start_out(g, b):
            pltpu.async_copy(
                bufs[b], out_hbm.at[pl.ds(base + g * _C, _C)], osems[b]
            )

        def wait_out(g, b):
            pltpu.make_async_copy(
                bufs[b], out_hbm.at[pl.ds(base + g * _C, _C)], osems[b]
            ).wait()

        def scale(b):
            buf = bufs[b]

            @plsc.parallel_loop(0, _C, step=1, unroll=4)
            def _(r):
                for j in range(_D // 16):
                    sl = pl.ds(j * 16, 16)
                    buf[r, sl] = buf[r, sl] * _SCALE

        start_gather(0, 0)

        def step(i, carry):
            g = 2 * i

            @pl.when(i >= 1)
            def _():
                wait_out(g - 1, 1)

            start_gather(g + 1, 1)
            wait_gather(g, 0)
            scale(0)
            start_out(g, 0)

            @pl.when(g + 2 < g_chunks)
            def _():
                wait_out(g, 0)
                start_gather(g + 2, 0)

            wait_gather(g + 1, 1)
            scale(1)
            start_out(g + 1, 1)
            return carry

        lax.fori_loop(0, g_chunks // 2, step, 0)
        wait_out(g_chunks - 2, 0)
        wait_out(g_chunks - 1, 1)

    return k(idx, table)


def kernel(inputs, table):
    shape = inputs.shape
    n = inputs.size
    idx = inputs.reshape(32, -1, _C).astype(jnp.int32)
    out = _embed(n, idx, table)
    return out.reshape(*shape, _D)


# speedup vs baseline: 1.0565x; 1.0565x over previous
import functools

import jax
import jax.numpy as jnp
from jax import lax
from jax.experimental import pallas as pl
from jax.experimental.pallas import tpu as pltpu
from jax.experimental.pallas import tpu_sc as plsc

_SCALE = 11.313708498984761
_D = 128
_C = 64
_NBUF = 4


@functools.partial(jax.jit, static_argnums=(0,))
def _embed(n_rows, idx, table):
    info = plsc.get_sparse_core_info()
    num_cores, num_subcores = info.num_cores, info.num_subcores
    nw = num_cores * num_subcores
    b_per_w = n_rows // nw
    g_chunks = b_per_w // _C

    mesh = plsc.VectorSubcoreMesh(core_axis_name="c", subcore_axis_name="s")

    @functools.partial(
        pl.kernel,
        mesh=mesh,
        out_type=jax.ShapeDtypeStruct((n_rows, _D), jnp.float32),
        scratch_types=[
            pltpu.VMEM((g_chunks, _C), jnp.int32),
        ]
        + [pltpu.VMEM((_C, _D), jnp.float32) for _ in range(_NBUF)]
        + [pltpu.SemaphoreType.DMA for _ in range(2 * _NBUF)],
    )
    def k(idx_hbm, table_hbm, out_hbm, idx_v, *bufs_and_sems):
        bufs = bufs_and_sems[:_NBUF]
        gsems = bufs_and_sems[_NBUF : 2 * _NBUF]
        osems = bufs_and_sems[2 * _NBUF :]

        wid = lax.axis_index("s") * num_cores + lax.axis_index("c")
        base = wid * b_per_w
        pltpu.sync_copy(idx_hbm.at[wid], idx_v)

        def start_gather(g, b):
            pltpu.async_copy(table_hbm.at[idx_v.at[g]], bufs[b], gsems[b])

        def wait_gather(g, b):
            pltpu.make_async_copy(
                table_hbm.at[idx_v.at[g]], bufs[b], gsems[b]
            ).wait()

        def start_out(g, b):
            pltpu.async_copy(
                bufs[b], out_hbm.at[pl.ds(base + g * _C, _C)], osems[b]
            )

        def wait_out(g, b):
            pltpu.make_async_copy(
                bufs[b], out_hbm.at[pl.ds(base + g * _C, _C)], osems[b]
            ).wait()

        def scale(b):
            buf = bufs[b]

            @plsc.parallel_loop(0, _C, step=1, unroll=4)
            def _(r):
                for j in range(_D // 16):
                    sl = pl.ds(j * 16, 16)
                    buf[r, sl] = buf[r, sl] * _SCALE

        for g in range(_NBUF - 1):
            start_gather(g, g)

        def step(i, carry):
            for u in range(_NBUF):
                g = _NBUF * i + u
                nb = (u + _NBUF - 1) % _NBUF

                @pl.when(g + _NBUF - 1 < g_chunks)
                def _(g=g, nb=nb):
                    @pl.when(g >= 1)
                    def _():
                        wait_out(g - 1, nb)

                    start_gather(g + _NBUF - 1, nb)

                wait_gather(g, u)
                scale(u)
                start_out(g, u)
            return carry

        lax.fori_loop(0, g_chunks // _NBUF, step, 0)
        for g in range(g_chunks - _NBUF, g_chunks):
            wait_out(g, g % _NBUF)

    return k(idx, table)


def kernel(inputs, table):
    shape = inputs.shape
    n = inputs.size
    idx = inputs.reshape(32, -1, _C).astype(jnp.int32)
    out = _embed(n, idx, table)
    return out.reshape(*shape, _D)


# speedup vs baseline: 1.0584x; 1.0018x over previous
import functools

import jax
import jax.numpy as jnp
from jax import lax
from jax.experimental import pallas as pl
from jax.experimental.pallas import tpu as pltpu
from jax.experimental.pallas import tpu_sc as plsc

_SCALE = 11.313708498984761
_D = 128
_C = 128
_NBUF = 4


@functools.partial(jax.jit, static_argnums=(0,))
def _embed(n_rows, idx, table):
    info = plsc.get_sparse_core_info()
    num_cores, num_subcores = info.num_cores, info.num_subcores
    nw = num_cores * num_subcores
    b_per_w = n_rows // nw
    g_chunks = b_per_w // _C
    g_main = (g_chunks // _NBUF) * _NBUF

    mesh = plsc.VectorSubcoreMesh(core_axis_name="c", subcore_axis_name="s")

    @functools.partial(
        pl.kernel,
        mesh=mesh,
        out_type=jax.ShapeDtypeStruct((n_rows, _D), jnp.float32),
        scratch_types=[
            pltpu.VMEM((g_chunks, _C), jnp.int32),
        ]
        + [pltpu.VMEM((_C, _D), jnp.float32) for _ in range(_NBUF)]
        + [pltpu.SemaphoreType.DMA for _ in range(2 * _NBUF)],
    )
    def k(idx_hbm, table_hbm, out_hbm, idx_v, *bufs_and_sems):
        bufs = bufs_and_sems[:_NBUF]
        gsems = bufs_and_sems[_NBUF : 2 * _NBUF]
        osems = bufs_and_sems[2 * _NBUF :]

        wid = lax.axis_index("s") * num_cores + lax.axis_index("c")
        base = wid * b_per_w
        pltpu.sync_copy(idx_hbm.at[wid], idx_v)

        def start_gather(g, b):
            pltpu.async_copy(table_hbm.at[idx_v.at[g]], bufs[b], gsems[b])

        def wait_gather(g, b):
            pltpu.make_async_copy(
                table_hbm.at[idx_v.at[g]], bufs[b], gsems[b]
            ).wait()

        def start_out(g, b):
            pltpu.async_copy(
                bufs[b], out_hbm.at[pl.ds(base + g * _C, _C)], osems[b]
            )

        def wait_out(g, b):
            pltpu.make_async_copy(
                bufs[b], out_hbm.at[pl.ds(base + g * _C, _C)], osems[b]
            ).wait()

        def scale(b):
            buf = bufs[b]

            @plsc.parallel_loop(0, _C, step=1, unroll=4)
            def _(r):
                for j in range(_D // 16):
                    sl = pl.ds(j * 16, 16)
                    buf[r, sl] = buf[r, sl] * _SCALE

        for g in range(_NBUF - 1):
            start_gather(g, g)

        def body(g, u, dynamic):
            f_ok = (g + _NBUF - 1 < g_chunks) if not dynamic else None
            nb = (u + _NBUF - 1) % _NBUF

            def prefetch():
                @pl.when(g >= 1) if dynamic else _run_if(g >= 1)
                def _():
                    wait_out(g - 1, nb)

                start_gather(g + _NBUF - 1, nb)

            if dynamic:

                @pl.when(g + _NBUF - 1 < g_chunks)
                def _():
                    prefetch()

            elif f_ok:
                prefetch()

            wait_gather(g, u)
            scale(u)
            start_out(g, u)

        def _run_if(cond):
            def deco(fn):
                if cond:
                    fn()

            return deco

        def step(i, carry):
            for u in range(_NBUF):
                body(_NBUF * i + u, u, dynamic=True)
            return carry

        lax.fori_loop(0, g_main // _NBUF, step, 0)
        for g in range(g_main, g_chunks):
            body(g, g % _NBUF, dynamic=False)
        for g in range(g_chunks - _NBUF, g_chunks):
            wait_out(g, g % _NBUF)

    return k(idx, table)


def kernel(inputs, table):
    shape = inputs.shape
    n = inputs.size
    idx = inputs.reshape(32, -1, _C).astype(jnp.int32)
    out = _embed(n, idx, table)
    return out.reshape(*shape, _D)
